# Initial kernel scaffold; baseline (speedup 1.0000x reference)
import functools

import jax
import jax.numpy as jnp
from jax import lax
from jax.experimental import pallas as pl
from jax.experimental.pallas import tpu as pltpu, tpu_sc as plsc

N = 100000
E = 6400000
IN_DIM = 10
HID = 64
NUM_CLASSES = 4

NC = 2
NS = 16
KW = 80
KR = 25
K = KW * KR
EROWS = E // KW

ROWS_PER_TILE_FULL = E // NS // KW
CHUNKS_FULL = ROWS_PER_TILE_FULL // KR
ROWS_PER_WORKER = E // (NC * NS) // KW
CHUNKS_HALF = ROWS_PER_WORKER // KR

NPT = N // NS
HIST = 100096
HPT = HIST // NS

_MESH = plsc.VectorSubcoreMesh(
    core_axis_name="c", subcore_axis_name="s", num_cores=NC, num_subcores=NS)


def _deg_body(ed, zeros_h, ones_h, out, colbuf, ones_v, hist, sem):
  del sem
  c = lax.axis_index("c")
  s = lax.axis_index("s")
  pltpu.sync_copy(ones_h, ones_v)
  pltpu.sync_copy(zeros_h.at[pl.ds(s * HPT, HPT)], hist.at[pl.ds(s * HPT, HPT)])
  plsc.subcore_barrier()
  base = (c * NS + s) * ROWS_PER_WORKER

  def body(i, carry):
    r0 = base + i * KR
    pltpu.sync_copy(ed.at[1].at[pl.ds(r0, KR)], colbuf)
    pltpu.sync_copy(ones_v, hist.at[colbuf], add=True)
    return carry

  lax.fori_loop(0, CHUNKS_HALF, body, 0)
  plsc.subcore_barrier()
  pltpu.sync_copy(hist.at[pl.ds(s * HPT, HPT)],
                  out.at[c].at[pl.ds(s * HPT, HPT)])


_deg_call = pl.kernel(
    _deg_body,
    out_type=jax.ShapeDtypeStruct((NC, HIST), jnp.float32),
    mesh=_MESH,
    scratch_types=[
        pltpu.VMEM((KR, KW), jnp.int32),
        pltpu.VMEM((KR, KW), jnp.float32),
        pltpu.VMEM_SHARED((HIST,), jnp.float32),
        pltpu.SemaphoreType.DMA,
    ],
)


def _spmm64_body(ed, y, zeros_h, out, colbuf, rowbuf, stage, acc, sem):
  c = lax.axis_index("c")
  s = lax.axis_index("s")
  for j in range(2):
    g = 2 * c + j
    pltpu.sync_copy(zeros_h, acc.at[pl.ds(s * NPT, NPT)])
    plsc.subcore_barrier()
    base = s * ROWS_PER_TILE_FULL

    def body(i, carry):
      r0 = base + i * KR
      pltpu.sync_copy(ed.at[0].at[pl.ds(r0, KR)], rowbuf)
      pltpu.sync_copy(ed.at[1].at[pl.ds(r0, KR)], colbuf)
      pltpu.async_copy(y.at[g].at[rowbuf], stage, sem).wait()
      pltpu.sync_copy(stage, acc.at[colbuf], add=True)
      return carry

    lax.fori_loop(0, CHUNKS_FULL, body, 0)
    plsc.subcore_barrier()
    pltpu.sync_copy(acc.at[pl.ds(s * NPT, NPT)],
                    out.at[g].at[pl.ds(s * NPT, NPT)])
    plsc.subcore_barrier()


_spmm64_call = pl.kernel(
    _spmm64_body,
    out_type=jax.ShapeDtypeStruct((4, N, 16), jnp.float32),
    mesh=_MESH,
    scratch_types=[
        pltpu.VMEM((KR, KW), jnp.int32),
        pltpu.VMEM((KR, KW), jnp.int32),
        pltpu.VMEM((K, 16), jnp.float32),
        pltpu.VMEM_SHARED((N, 16), jnp.float32),
        pltpu.SemaphoreType.DMA,
    ],
)


def _spmm16_body(ed, y, zeros_h, out, colbuf, rowbuf, stage, acc, sem):
  c = lax.axis_index("c")
  s = lax.axis_index("s")
  pltpu.sync_copy(zeros_h, acc.at[pl.ds(s * NPT, NPT)])
  plsc.subcore_barrier()
  base = (c * NS + s) * ROWS_PER_WORKER

  def body(i, carry):
    r0 = base + i * KR
    pltpu.sync_copy(ed.at[0].at[pl.ds(r0, KR)], rowbuf)
    pltpu.sync_copy(ed.at[1].at[pl.ds(r0, KR)], colbuf)
    pltpu.async_copy(y.at[rowbuf], stage, sem).wait()
    pltpu.sync_copy(stage, acc.at[colbuf], add=True)
    return carry

  lax.fori_loop(0, CHUNKS_HALF, body, 0)
  plsc.subcore_barrier()
  pltpu.sync_copy(acc.at[pl.ds(s * NPT, NPT)],
                  out.at[c].at[pl.ds(s * NPT, NPT)])


_spmm16_call = pl.kernel(
    _spmm16_body,
    out_type=jax.ShapeDtypeStruct((NC, N, 16), jnp.float32),
    mesh=_MESH,
    scratch_types=[
        pltpu.VMEM((KR, KW), jnp.int32),
        pltpu.VMEM((KR, KW), jnp.int32),
        pltpu.VMEM((K, 16), jnp.float32),
        pltpu.VMEM_SHARED((N, 16), jnp.float32),
        pltpu.SemaphoreType.DMA,
    ],
)


_R = 2000
_GRID = N // _R


def _tcA_kernel(pT, x, w1, dis_ref, y1_ref):
  deg = pT[:, 0:1] + pT[:, 1:2] + 1.0
  dis = lax.rsqrt(deg)
  dis_ref[...] = dis
  xw = jnp.dot(x[...], w1[...], preferred_element_type=jnp.float32)
  for g in range(4):
    y1_ref[g] = xw[:, g * 16:(g + 1) * 16] * dis


def _tcA(pT, x, w1):
  return pl.pallas_call(
      _tcA_kernel,
      grid=(_GRID,),
      in_specs=[
          pl.BlockSpec((_R, NC), lambda i: (i, 0)),
          pl.BlockSpec((_R, IN_DIM), lambda i: (i, 0)),
          pl.BlockSpec((IN_DIM, HID), lambda i: (0, 0)),
      ],
      out_specs=[
          pl.BlockSpec((_R, 1), lambda i: (i, 0)),
          pl.BlockSpec((4, _R, 16), lambda i: (0, i, 0)),
      ],
      out_shape=[
          jax.ShapeDtypeStruct((N, 1), jnp.float32),
          jax.ShapeDtypeStruct((4, N, 16), jnp.float32),
      ],
  )(pT, x, w1)


def _tcMid_kernel(s_in, y_in, dis_in, b_in, w_in, ynext_ref):
  dis = dis_in[...]
  h = jnp.concatenate([s_in[g] + y_in[g] for g in range(4)], axis=1)
  h = jnp.maximum(h * dis + b_in[...], 0.0)
  xw = jnp.dot(h, w_in[...], preferred_element_type=jnp.float32)
  for g in range(4):
    ynext_ref[g] = xw[:, g * 16:(g + 1) * 16] * dis


def _tcMid(s_in, y_in, dis, b, w):
  return pl.pallas_call(
      _tcMid_kernel,
      grid=(_GRID,),
      in_specs=[
          pl.BlockSpec((4, _R, 16), lambda i: (0, i, 0)),
          pl.BlockSpec((4, _R, 16), lambda i: (0, i, 0)),
          pl.BlockSpec((_R, 1), lambda i: (i, 0)),
          pl.BlockSpec((1, HID), lambda i: (0, 0)),
          pl.BlockSpec((HID, HID), lambda i: (0, 0)),
      ],
      out_specs=pl.BlockSpec((4, _R, 16), lambda i: (0, i, 0)),
      out_shape=jax.ShapeDtypeStruct((4, N, 16), jnp.float32),
  )(s_in, y_in, dis, b, w)


def _tcC_kernel(s_in, y_in, dis_in, b_in, w_in, y3_ref):
  dis = dis_in[...]
  h = jnp.concatenate([s_in[g] + y_in[g] for g in range(4)], axis=1)
  h = jnp.maximum(h * dis + b_in[...], 0.0)
  xw = jnp.dot(h, w_in[...], preferred_element_type=jnp.float32)
  y3_ref[...] = jnp.concatenate(
      [xw * dis, jnp.zeros((_R, 16 - NUM_CLASSES), jnp.float32)], axis=1)


def _tcC(s_in, y_in, dis, b, w):
  return pl.pallas_call(
      _tcC_kernel,
      grid=(_GRID,),
      in_specs=[
          pl.BlockSpec((4, _R, 16), lambda i: (0, i, 0)),
          pl.BlockSpec((4, _R, 16), lambda i: (0, i, 0)),
          pl.BlockSpec((_R, 1), lambda i: (i, 0)),
          pl.BlockSpec((1, HID), lambda i: (0, 0)),
          pl.BlockSpec((HID, NUM_CLASSES), lambda i: (0, 0)),
      ],
      out_specs=pl.BlockSpec((_R, 16), lambda i: (i, 0)),
      out_shape=jax.ShapeDtypeStruct((N, 16), jnp.float32),
  )(s_in, y_in, dis, b, w)


def _tcD_kernel(t_in, y3_in, dis_in, b_in, out_ref):
  z = (t_in[0, :, 0:NUM_CLASSES] + t_in[1, :, 0:NUM_CLASSES]
       + y3_in[:, 0:NUM_CLASSES])
  z = z * dis_in[...] + b_in[...]
  m = jnp.max(z, axis=1, keepdims=True)
  u = z - m
  out_ref[...] = u - jnp.log(jnp.sum(jnp.exp(u), axis=1, keepdims=True))


def _tcD(t, y3, dis, b):
  return pl.pallas_call(
      _tcD_kernel,
      grid=(_GRID,),
      in_specs=[
          pl.BlockSpec((NC, _R, 16), lambda i: (0, i, 0)),
          pl.BlockSpec((_R, 16), lambda i: (i, 0)),
          pl.BlockSpec((_R, 1), lambda i: (i, 0)),
          pl.BlockSpec((1, NUM_CLASSES), lambda i: (0, 0)),
      ],
      out_specs=pl.BlockSpec((_R, NUM_CLASSES), lambda i: (i, 0)),
      out_shape=jax.ShapeDtypeStruct((N, NUM_CLASSES), jnp.float32),
  )(t, y3, dis, b)


def kernel(x, edge_index, W1, b1, W2, b2, W3, b3):
  ed = edge_index.reshape(2, EROWS, KW)
  zeros_hist = jnp.zeros((HIST,), jnp.float32)
  zeros_acc = jnp.zeros((NPT, 16), jnp.float32)
  ones_chunk = jnp.ones((KR, KW), jnp.float32)

  p = _deg_call(ed, zeros_hist, ones_chunk)
  dis, y1 = _tcA(p.T[:N], x, W1)
  s1 = _spmm64_call(ed, y1, zeros_acc)
  y2 = _tcMid(s1, y1, dis, b1.reshape(1, HID), W2)
  s2 = _spmm64_call(ed, y2, zeros_acc)
  y3 = _tcC(s2, y2, dis, b2.reshape(1, HID), W3)
  t = _spmm16_call(ed, y3, zeros_acc)
  return _tcD(t, y3, dis, b3.reshape(1, NUM_CLASSES))


# speedup vs baseline: 24.9161x; 24.9161x over previous
import jax
import jax.numpy as jnp
from jax import lax
from jax.experimental import pallas as pl
from jax.experimental.pallas import tpu as pltpu, tpu_sc as plsc

N = 100000
E = 6400000
IN_DIM = 10
HID = 64
NUM_CLASSES = 4

NC = 2
NS = 16
NW = NC * NS
KD = 2048
DEG_T = (E // KD) // NW
DEG_REM = (E // KD) % NW
K = 1024
NCHUNKS = E // K
FULL_T = NCHUNKS // NS
FULL_REM = NCHUNKS % NS
HALF_T = NCHUNKS // NW
HALF_REM = NCHUNKS % NW

NP = 100352
NPT = NP // NS

_MESH = plsc.VectorSubcoreMesh(
    core_axis_name="c", subcore_axis_name="s", num_cores=NC, num_subcores=NS)
_SC_PARAMS = pltpu.CompilerParams(use_tc_tiling_on_sc=False)


def _deg_body(ed, zeros_h, ones_h, out, colbuf, ones_v, hist, sem):
  del sem
  c = lax.axis_index("c")
  s = lax.axis_index("s")
  w = c * NS + s
  pltpu.sync_copy(ones_h, ones_v)
  pltpu.sync_copy(zeros_h.at[pl.ds(s * NPT, NPT)], hist.at[pl.ds(s * NPT, NPT)])
  plsc.subcore_barrier()

  def step(chunk):
    e0 = pl.multiple_of(chunk * KD, KD)
    pltpu.sync_copy(ed.at[1].at[pl.ds(e0, KD)], colbuf)
    pltpu.sync_copy(ones_v, hist.at[colbuf], add=True)

  def body(t, carry):
    step(w + NW * t)
    return carry

  lax.fori_loop(0, DEG_T, body, 0)

  @pl.when(w < DEG_REM)
  def _():
    step(NW * DEG_T + w)

  plsc.subcore_barrier()
  pltpu.sync_copy(hist.at[pl.ds(s * NPT, NPT)],
                  out.at[c].at[pl.ds(s * NPT, NPT)])


_deg_call = pl.kernel(
    _deg_body,
    compiler_params=_SC_PARAMS,
    out_type=jax.ShapeDtypeStruct((NC, NP), jnp.float32),
    mesh=_MESH,
    scratch_types=[
        pltpu.VMEM((KD,), jnp.int32),
        pltpu.VMEM((KD,), jnp.float32),
        pltpu.VMEM_SHARED((NP,), jnp.float32),
        pltpu.SemaphoreType.DMA,
    ],
)


def _spmm64_body(ed, y, zeros_h, out, colbuf, rowbuf, stage, acc, sem):
  c = lax.axis_index("c")
  s = lax.axis_index("s")
  for j in range(2):
    g = 2 * c + j
    pltpu.sync_copy(zeros_h, acc.at[pl.ds(s * NPT, NPT)])
    plsc.subcore_barrier()

    def step(chunk):
      e0 = pl.multiple_of(chunk * K, K)
      pltpu.sync_copy(ed.at[0].at[pl.ds(e0, K)], rowbuf)
      pltpu.sync_copy(ed.at[1].at[pl.ds(e0, K)], colbuf)
      pltpu.async_copy(y.at[g].at[rowbuf], stage, sem).wait()
      pltpu.sync_copy(stage, acc.at[colbuf], add=True)

    def body(t, carry):
      step(s + NS * t)
      return carry

    lax.fori_loop(0, FULL_T, body, 0)

    @pl.when(s < FULL_REM)
    def _():
      step(NS * FULL_T + s)

    plsc.subcore_barrier()
    pltpu.sync_copy(acc.at[pl.ds(s * NPT, NPT)],
                    out.at[g].at[pl.ds(s * NPT, NPT)])
    plsc.subcore_barrier()


_spmm64_call = pl.kernel(
    _spmm64_body,
    compiler_params=_SC_PARAMS,
    out_type=jax.ShapeDtypeStruct((4, NP, 16), jnp.float32),
    mesh=_MESH,
    scratch_types=[
        pltpu.VMEM((K,), jnp.int32),
        pltpu.VMEM((K,), jnp.int32),
        pltpu.VMEM((K, 16), jnp.float32),
        pltpu.VMEM_SHARED((NP, 16), jnp.float32),
        pltpu.SemaphoreType.DMA,
    ],
)


def _spmm16_body(ed, y, zeros_h, out, colbuf, rowbuf, stage, acc, sem):
  c = lax.axis_index("c")
  s = lax.axis_index("s")
  w = c * NS + s
  pltpu.sync_copy(zeros_h, acc.at[pl.ds(s * NPT, NPT)])
  plsc.subcore_barrier()

  def step(chunk):
    e0 = pl.multiple_of(chunk * K, K)
    pltpu.sync_copy(ed.at[0].at[pl.ds(e0, K)], rowbuf)
    pltpu.sync_copy(ed.at[1].at[pl.ds(e0, K)], colbuf)
    pltpu.async_copy(y.at[rowbuf], stage, sem).wait()
    pltpu.sync_copy(stage, acc.at[colbuf], add=True)

  def body(t, carry):
    step(w + NW * t)
    return carry

  lax.fori_loop(0, HALF_T, body, 0)

  @pl.when(w < HALF_REM)
  def _():
    step(NW * HALF_T + w)

  plsc.subcore_barrier()
  pltpu.sync_copy(acc.at[pl.ds(s * NPT, NPT)],
                  out.at[c].at[pl.ds(s * NPT, NPT)])


_spmm16_call = pl.kernel(
    _spmm16_body,
    compiler_params=_SC_PARAMS,
    out_type=jax.ShapeDtypeStruct((NC, NP, 16), jnp.float32),
    mesh=_MESH,
    scratch_types=[
        pltpu.VMEM((K,), jnp.int32),
        pltpu.VMEM((K,), jnp.int32),
        pltpu.VMEM((K, 16), jnp.float32),
        pltpu.VMEM_SHARED((NP, 16), jnp.float32),
        pltpu.SemaphoreType.DMA,
    ],
)


_R = 2000
_GRID = N // _R


def _tcA_kernel(pT, x, w1, dis_ref, y1_ref):
  deg = pT[:, 0:1] + pT[:, 1:2] + 1.0
  dis = lax.rsqrt(deg)
  dis_ref[...] = dis
  xw = jnp.dot(x[...], w1[...], preferred_element_type=jnp.float32)
  for g in range(4):
    y1_ref[g] = xw[:, g * 16:(g + 1) * 16] * dis


def _tcA(pT, x, w1):
  return pl.pallas_call(
      _tcA_kernel,
      grid=(_GRID,),
      in_specs=[
          pl.BlockSpec((_R, NC), lambda i: (i, 0)),
          pl.BlockSpec((_R, IN_DIM), lambda i: (i, 0)),
          pl.BlockSpec((IN_DIM, HID), lambda i: (0, 0)),
      ],
      out_specs=[
          pl.BlockSpec((_R, 1), lambda i: (i, 0)),
          pl.BlockSpec((4, _R, 16), lambda i: (0, i, 0)),
      ],
      out_shape=[
          jax.ShapeDtypeStruct((N, 1), jnp.float32),
          jax.ShapeDtypeStruct((4, NP, 16), jnp.float32),
      ],
  )(pT, x, w1)


def _tcMid_kernel(s_in, y_in, dis_in, b_in, w_in, ynext_ref):
  dis = dis_in[...]
  h = jnp.concatenate([s_in[g] + y_in[g] for g in range(4)], axis=1)
  h = jnp.maximum(h * dis + b_in[...], 0.0)
  xw = jnp.dot(h, w_in[...], preferred_element_type=jnp.float32)
  for g in range(4):
    ynext_ref[g] = xw[:, g * 16:(g + 1) * 16] * dis


def _tcMid(s_in, y_in, dis, b, w):
  return pl.pallas_call(
      _tcMid_kernel,
      grid=(_GRID,),
      in_specs=[
          pl.BlockSpec((4, _R, 16), lambda i: (0, i, 0)),
          pl.BlockSpec((4, _R, 16), lambda i: (0, i, 0)),
          pl.BlockSpec((_R, 1), lambda i: (i, 0)),
          pl.BlockSpec((1, HID), lambda i: (0, 0)),
          pl.BlockSpec((HID, HID), lambda i: (0, 0)),
      ],
      out_specs=pl.BlockSpec((4, _R, 16), lambda i: (0, i, 0)),
      out_shape=jax.ShapeDtypeStruct((4, NP, 16), jnp.float32),
  )(s_in, y_in, dis, b, w)


def _tcC_kernel(s_in, y_in, dis_in, b_in, w_in, y3_ref):
  dis = dis_in[...]
  h = jnp.concatenate([s_in[g] + y_in[g] for g in range(4)], axis=1)
  h = jnp.maximum(h * dis + b_in[...], 0.0)
  xw = jnp.dot(h, w_in[...], preferred_element_type=jnp.float32)
  y3_ref[...] = jnp.concatenate(
      [xw * dis, jnp.zeros((_R, 16 - NUM_CLASSES), jnp.float32)], axis=1)


def _tcC(s_in, y_in, dis, b, w):
  return pl.pallas_call(
      _tcC_kernel,
      grid=(_GRID,),
      in_specs=[
          pl.BlockSpec((4, _R, 16), lambda i: (0, i, 0)),
          pl.BlockSpec((4, _R, 16), lambda i: (0, i, 0)),
          pl.BlockSpec((_R, 1), lambda i: (i, 0)),
          pl.BlockSpec((1, HID), lambda i: (0, 0)),
          pl.BlockSpec((HID, NUM_CLASSES), lambda i: (0, 0)),
      ],
      out_specs=pl.BlockSpec((_R, 16), lambda i: (i, 0)),
      out_shape=jax.ShapeDtypeStruct((NP, 16), jnp.float32),
  )(s_in, y_in, dis, b, w)


def _tcD_kernel(t_in, y3_in, dis_in, b_in, out_ref):
  z = (t_in[0, :, 0:NUM_CLASSES] + t_in[1, :, 0:NUM_CLASSES]
       + y3_in[:, 0:NUM_CLASSES])
  z = z * dis_in[...] + b_in[...]
  m = jnp.max(z, axis=1, keepdims=True)
  u = z - m
  out_ref[...] = u - jnp.log(jnp.sum(jnp.exp(u), axis=1, keepdims=True))


def _tcD(t, y3, dis, b):
  return pl.pallas_call(
      _tcD_kernel,
      grid=(_GRID,),
      in_specs=[
          pl.BlockSpec((NC, _R, 16), lambda i: (0, i, 0)),
          pl.BlockSpec((_R, 16), lambda i: (i, 0)),
          pl.BlockSpec((_R, 1), lambda i: (i, 0)),
          pl.BlockSpec((1, NUM_CLASSES), lambda i: (0, 0)),
      ],
      out_specs=pl.BlockSpec((_R, NUM_CLASSES), lambda i: (i, 0)),
      out_shape=jax.ShapeDtypeStruct((N, NUM_CLASSES), jnp.float32),
  )(t, y3, dis, b)


def kernel(x, edge_index, W1, b1, W2, b2, W3, b3):
  zeros_hist = jnp.zeros((NP,), jnp.float32)
  zeros_acc = jnp.zeros((NPT, 16), jnp.float32)
  ones_chunk = jnp.ones((KD,), jnp.float32)

  p = _deg_call(edge_index, zeros_hist, ones_chunk)
  dis, y1 = _tcA(p.T[:N], x, W1)
  s1 = _spmm64_call(edge_index, y1, zeros_acc)
  y2 = _tcMid(s1, y1, dis, b1.reshape(1, HID), W2)
  s2 = _spmm64_call(edge_index, y2, zeros_acc)
  y3 = _tcC(s2, y2, dis, b2.reshape(1, HID), W3)
  t = _spmm16_call(edge_index, y3, zeros_acc)
  return _tcD(t, y3, dis, b3.reshape(1, NUM_CLASSES))


# speedup vs baseline: 35.0258x; 1.4057x over previous
import jax
import jax.numpy as jnp
from jax import lax
from jax.experimental import pallas as pl
from jax.experimental.pallas import tpu as pltpu, tpu_sc as plsc

N = 100000
E = 6400000
IN_DIM = 10
HID = 64
NUM_CLASSES = 4

NC = 2
NS = 16
NW = NC * NS
KD = 2048
DEG_T = (E // KD) // NW
DEG_REM = (E // KD) % NW
K = 640
NCHUNKS = E // K
FULL_T = NCHUNKS // NS
FULL_PAIRS = FULL_T // 2
HALF_T = NCHUNKS // NW
HALF_REM = NCHUNKS % NW

NP = 100352
NPT = NP // NS

_MESH = plsc.VectorSubcoreMesh(
    core_axis_name="c", subcore_axis_name="s", num_cores=NC, num_subcores=NS)
_SC_PARAMS = pltpu.CompilerParams(use_tc_tiling_on_sc=False)


def _deg_body(ed, zeros_h, ones_h, out, colbuf, ones_v, hist, sem):
  del sem
  c = lax.axis_index("c")
  s = lax.axis_index("s")
  w = c * NS + s
  pltpu.sync_copy(ones_h, ones_v)
  pltpu.sync_copy(zeros_h.at[pl.ds(s * NPT, NPT)], hist.at[pl.ds(s * NPT, NPT)])
  plsc.subcore_barrier()

  def step(chunk):
    e0 = pl.multiple_of(chunk * KD, KD)
    pltpu.sync_copy(ed.at[1].at[pl.ds(e0, KD)], colbuf)
    pltpu.sync_copy(ones_v, hist.at[colbuf], add=True)

  def body(t, carry):
    step(w + NW * t)
    return carry

  lax.fori_loop(0, DEG_T, body, 0)

  @pl.when(w < DEG_REM)
  def _():
    step(NW * DEG_T + w)

  plsc.subcore_barrier()
  pltpu.sync_copy(hist.at[pl.ds(s * NPT, NPT)],
                  out.at[c].at[pl.ds(s * NPT, NPT)])


_deg_call = pl.kernel(
    _deg_body,
    compiler_params=_SC_PARAMS,
    out_type=jax.ShapeDtypeStruct((NC, NP), jnp.float32),
    mesh=_MESH,
    scratch_types=[
        pltpu.VMEM((KD,), jnp.int32),
        pltpu.VMEM((KD,), jnp.float32),
        pltpu.VMEM_SHARED((NP,), jnp.float32),
        pltpu.SemaphoreType.DMA,
    ],
)


def _spmm64_body(ed, y, zeros_h, out, cb0, rb0, st0, cb1, rb1, st1, acc,
                 sr0, sc0, sg0, ss0, sr1, sc1, sg1, ss1):
  c = lax.axis_index("c")
  s = lax.axis_index("s")
  for j in range(2):
    g = 2 * c + j
    pltpu.sync_copy(zeros_h, acc.at[pl.ds(s * NPT, NPT)])
    plsc.subcore_barrier()

    def pair(t, carry):
      e0 = pl.multiple_of((s + NS * 2 * t) * K, K)
      e1 = pl.multiple_of((s + NS * (2 * t + 1)) * K, K)
      hr0 = pltpu.async_copy(ed.at[0].at[pl.ds(e0, K)], rb0, sr0)
      hc0 = pltpu.async_copy(ed.at[1].at[pl.ds(e0, K)], cb0, sc0)
      hr1 = pltpu.async_copy(ed.at[0].at[pl.ds(e1, K)], rb1, sr1)
      hc1 = pltpu.async_copy(ed.at[1].at[pl.ds(e1, K)], cb1, sc1)
      hr0.wait()
      hg0 = pltpu.async_copy(y.at[g].at[rb0], st0, sg0)
      hr1.wait()
      hg1 = pltpu.async_copy(y.at[g].at[rb1], st1, sg1)
      hg0.wait()
      hc0.wait()
      hs0 = pltpu.async_copy(st0, acc.at[cb0], ss0, add=True)
      hg1.wait()
      hc1.wait()
      hs1 = pltpu.async_copy(st1, acc.at[cb1], ss1, add=True)
      hs0.wait()
      hs1.wait()
      return carry

    lax.fori_loop(0, FULL_PAIRS, pair, 0)

    e0 = pl.multiple_of((s + NS * (FULL_T - 1)) * K, K)
    pltpu.async_copy(ed.at[0].at[pl.ds(e0, K)], rb0, sr0).wait()
    pltpu.async_copy(ed.at[1].at[pl.ds(e0, K)], cb0, sc0).wait()
    pltpu.async_copy(y.at[g].at[rb0], st0, sg0).wait()
    pltpu.async_copy(st0, acc.at[cb0], ss0, add=True).wait()

    plsc.subcore_barrier()
    pltpu.sync_copy(acc.at[pl.ds(s * NPT, NPT)],
                    out.at[g].at[pl.ds(s * NPT, NPT)])
    plsc.subcore_barrier()


_spmm64_call = pl.kernel(
    _spmm64_body,
    compiler_params=_SC_PARAMS,
    out_type=jax.ShapeDtypeStruct((4, NP, 16), jnp.float32),
    mesh=_MESH,
    scratch_types=[
        pltpu.VMEM((K,), jnp.int32),
        pltpu.VMEM((K,), jnp.int32),
        pltpu.VMEM((K, 16), jnp.float32),
        pltpu.VMEM((K,), jnp.int32),
        pltpu.VMEM((K,), jnp.int32),
        pltpu.VMEM((K, 16), jnp.float32),
        pltpu.VMEM_SHARED((NP, 16), jnp.float32),
    ] + [pltpu.SemaphoreType.DMA] * 8,
)


def _spmm16_body(ed, y, zeros_h, out, cb0, rb0, st0, cb1, rb1, st1, acc,
                 sr0, sc0, sg0, ss0, sr1, sc1, sg1, ss1):
  c = lax.axis_index("c")
  s = lax.axis_index("s")
  w = c * NS + s
  pltpu.sync_copy(zeros_h, acc.at[pl.ds(s * NPT, NPT)])
  plsc.subcore_barrier()

  def pair(t, carry):
    e0 = pl.multiple_of((w + NW * 2 * t) * K, K)
    e1 = pl.multiple_of((w + NW * (2 * t + 1)) * K, K)
    hr0 = pltpu.async_copy(ed.at[0].at[pl.ds(e0, K)], rb0, sr0)
    hc0 = pltpu.async_copy(ed.at[1].at[pl.ds(e0, K)], cb0, sc0)
    hr1 = pltpu.async_copy(ed.at[0].at[pl.ds(e1, K)], rb1, sr1)
    hc1 = pltpu.async_copy(ed.at[1].at[pl.ds(e1, K)], cb1, sc1)
    hr0.wait()
    hg0 = pltpu.async_copy(y.at[rb0], st0, sg0)
    hr1.wait()
    hg1 = pltpu.async_copy(y.at[rb1], st1, sg1)
    hg0.wait()
    hc0.wait()
    hs0 = pltpu.async_copy(st0, acc.at[cb0], ss0, add=True)
    hg1.wait()
    hc1.wait()
    hs1 = pltpu.async_copy(st1, acc.at[cb1], ss1, add=True)
    hs0.wait()
    hs1.wait()
    return carry

  lax.fori_loop(0, HALF_T // 2, pair, 0)

  @pl.when(w < HALF_REM)
  def _():
    e0 = pl.multiple_of((NW * HALF_T + w) * K, K)
    pltpu.async_copy(ed.at[0].at[pl.ds(e0, K)], rb0, sr0).wait()
    pltpu.async_copy(ed.at[1].at[pl.ds(e0, K)], cb0, sc0).wait()
    pltpu.async_copy(y.at[rb0], st0, sg0).wait()
    pltpu.async_copy(st0, acc.at[cb0], ss0, add=True).wait()

  plsc.subcore_barrier()
  pltpu.sync_copy(acc.at[pl.ds(s * NPT, NPT)],
                  out.at[c].at[pl.ds(s * NPT, NPT)])


_spmm16_call = pl.kernel(
    _spmm16_body,
    compiler_params=_SC_PARAMS,
    out_type=jax.ShapeDtypeStruct((NC, NP, 16), jnp.float32),
    mesh=_MESH,
    scratch_types=[
        pltpu.VMEM((K,), jnp.int32),
        pltpu.VMEM((K,), jnp.int32),
        pltpu.VMEM((K, 16), jnp.float32),
        pltpu.VMEM((K,), jnp.int32),
        pltpu.VMEM((K,), jnp.int32),
        pltpu.VMEM((K, 16), jnp.float32),
        pltpu.VMEM_SHARED((NP, 16), jnp.float32),
    ] + [pltpu.SemaphoreType.DMA] * 8,
)


_R = 2000
_GRID = N // _R


def _tcA_kernel(pT, x, w1, dis_ref, y1_ref):
  deg = pT[:, 0:1] + pT[:, 1:2] + 1.0
  dis = lax.rsqrt(deg)
  dis_ref[...] = dis
  xw = jnp.dot(x[...], w1[...], preferred_element_type=jnp.float32)
  for g in range(4):
    y1_ref[g] = xw[:, g * 16:(g + 1) * 16] * dis


def _tcA(pT, x, w1):
  return pl.pallas_call(
      _tcA_kernel,
      grid=(_GRID,),
      in_specs=[
          pl.BlockSpec((_R, NC), lambda i: (i, 0)),
          pl.BlockSpec((_R, IN_DIM), lambda i: (i, 0)),
          pl.BlockSpec((IN_DIM, HID), lambda i: (0, 0)),
      ],
      out_specs=[
          pl.BlockSpec((_R, 1), lambda i: (i, 0)),
          pl.BlockSpec((4, _R, 16), lambda i: (0, i, 0)),
      ],
      out_shape=[
          jax.ShapeDtypeStruct((N, 1), jnp.float32),
          jax.ShapeDtypeStruct((4, NP, 16), jnp.float32),
      ],
  )(pT, x, w1)


def _tcMid_kernel(s_in, y_in, dis_in, b_in, w_in, ynext_ref):
  dis = dis_in[...]
  h = jnp.concatenate([s_in[g] + y_in[g] for g in range(4)], axis=1)
  h = jnp.maximum(h * dis + b_in[...], 0.0)
  xw = jnp.dot(h, w_in[...], preferred_element_type=jnp.float32)
  for g in range(4):
    ynext_ref[g] = xw[:, g * 16:(g + 1) * 16] * dis


def _tcMid(s_in, y_in, dis, b, w):
  return pl.pallas_call(
      _tcMid_kernel,
      grid=(_GRID,),
      in_specs=[
          pl.BlockSpec((4, _R, 16), lambda i: (0, i, 0)),
          pl.BlockSpec((4, _R, 16), lambda i: (0, i, 0)),
          pl.BlockSpec((_R, 1), lambda i: (i, 0)),
          pl.BlockSpec((1, HID), lambda i: (0, 0)),
          pl.BlockSpec((HID, HID), lambda i: (0, 0)),
      ],
      out_specs=pl.BlockSpec((4, _R, 16), lambda i: (0, i, 0)),
      out_shape=jax.ShapeDtypeStruct((4, NP, 16), jnp.float32),
  )(s_in, y_in, dis, b, w)


def _tcC_kernel(s_in, y_in, dis_in, b_in, w_in, y3_ref):
  dis = dis_in[...]
  h = jnp.concatenate([s_in[g] + y_in[g] for g in range(4)], axis=1)
  h = jnp.maximum(h * dis + b_in[...], 0.0)
  xw = jnp.dot(h, w_in[...], preferred_element_type=jnp.float32)
  y3_ref[...] = jnp.concatenate(
      [xw * dis, jnp.zeros((_R, 16 - NUM_CLASSES), jnp.float32)], axis=1)


def _tcC(s_in, y_in, dis, b, w):
  return pl.pallas_call(
      _tcC_kernel,
      grid=(_GRID,),
      in_specs=[
          pl.BlockSpec((4, _R, 16), lambda i: (0, i, 0)),
          pl.BlockSpec((4, _R, 16), lambda i: (0, i, 0)),
          pl.BlockSpec((_R, 1), lambda i: (i, 0)),
          pl.BlockSpec((1, HID), lambda i: (0, 0)),
          pl.BlockSpec((HID, NUM_CLASSES), lambda i: (0, 0)),
      ],
      out_specs=pl.BlockSpec((_R, 16), lambda i: (i, 0)),
      out_shape=jax.ShapeDtypeStruct((NP, 16), jnp.float32),
  )(s_in, y_in, dis, b, w)


def _tcD_kernel(t_in, y3_in, dis_in, b_in, out_ref):
  z = (t_in[0, :, 0:NUM_CLASSES] + t_in[1, :, 0:NUM_CLASSES]
       + y3_in[:, 0:NUM_CLASSES])
  z = z * dis_in[...] + b_in[...]
  m = jnp.max(z, axis=1, keepdims=True)
  u = z - m
  out_ref[...] = u - jnp.log(jnp.sum(jnp.exp(u), axis=1, keepdims=True))


def _tcD(t, y3, dis, b):
  return pl.pallas_call(
      _tcD_kernel,
      grid=(_GRID,),
      in_specs=[
          pl.BlockSpec((NC, _R, 16), lambda i: (0, i, 0)),
          pl.BlockSpec((_R, 16), lambda i: (i, 0)),
          pl.BlockSpec((_R, 1), lambda i: (i, 0)),
          pl.BlockSpec((1, NUM_CLASSES), lambda i: (0, 0)),
      ],
      out_specs=pl.BlockSpec((_R, NUM_CLASSES), lambda i: (i, 0)),
      out_shape=jax.ShapeDtypeStruct((N, NUM_CLASSES), jnp.float32),
  )(t, y3, dis, b)


def kernel(x, edge_index, W1, b1, W2, b2, W3, b3):
  zeros_hist = jnp.zeros((NP,), jnp.float32)
  zeros_acc = jnp.zeros((NPT, 16), jnp.float32)
  ones_chunk = jnp.ones((KD,), jnp.float32)

  p = _deg_call(edge_index, zeros_hist, ones_chunk)
  dis, y1 = _tcA(p.T[:N], x, W1)
  s1 = _spmm64_call(edge_index, y1, zeros_acc)
  y2 = _tcMid(s1, y1, dis, b1.reshape(1, HID), W2)
  s2 = _spmm64_call(edge_index, y2, zeros_acc)
  y3 = _tcC(s2, y2, dis, b2.reshape(1, HID), W3)
  t = _spmm16_call(edge_index, y3, zeros_acc)
  return _tcD(t, y3, dis, b3.reshape(1, NUM_CLASSES))
